# baseline (speedup 1.0000x reference)
import functools

import jax
import jax.numpy as jnp
from jax import lax
from jax.experimental import pallas as pl
from jax.experimental.pallas import tpu as pltpu
from jax.experimental.pallas import tpu_sc as plsc

_NC = 2
_NS = 16
_ROWS = 400


_CHUNK = 80
_SBS = 10
_RING = 4
_LOOK = 3
_PAD_ROWS = 8


def _pad_edges(e, nw):
  epw = e // nw
  blk = _CHUNK * _SBS
  return -(-epw // blk) * blk


def _make_seg_sum(n, e, h):
  nw = _NC * _NS
  epw = _pad_edges(e, nw)
  chunk = _CHUNK
  steps = epw // chunk
  sbs = _SBS
  assert epw % (chunk * sbs) == 0
  spl = -(-n // _NS) // 8 * 8 + 8
  last = n - (_NS - 1) * spl
  assert last > 0 and last % 8 == 0

  nsb = steps // sbs

  def body(h_hbm, src_hbm, dst_hbm, zeros_hbm, out_hbm,
           acc, src_v, dst_v, *bufs_and_sems):
    cid = lax.axis_index("c")
    sid = lax.axis_index("s")
    row0 = pl.multiple_of(sid * spl, 8)

    def seed(rows):
      @pl.when(cid == 0)
      def _():
        pltpu.sync_copy(h_hbm.at[pl.ds(row0, rows)],
                        acc.at[pl.ds(row0, rows)])

      @pl.when(cid != 0)
      def _():
        pltpu.sync_copy(zeros_hbm.at[pl.ds(row0, rows)],
                        acc.at[pl.ds(row0, rows)])

    @pl.when(sid < _NS - 1)
    def _():
      seed(spl)

    @pl.when(sid == _NS - 1)
    def _():
      seed(last)

    plsc.subcore_barrier()

    wid = sid * _NC + cid
    ring, look = _RING, _LOOK
    bufs = bufs_and_sems[:ring]
    sems = bufs_and_sems[ring:ring + ring]
    sem_i = bufs_and_sems[ring + ring]

    def idx_row(ref, i):
      return ref.at[lax.rem(lax.div(i, sbs), 2), lax.rem(i, sbs)]

    def start_gather(i, k):
      pltpu.async_copy(h_hbm.at[idx_row(src_v, i)], bufs[k], sems[k])

    def wait_gather(i, k):
      pltpu.make_async_copy(h_hbm.at[idx_row(src_v, i)], bufs[k],
                            sems[k]).wait()

    def start_scatter(i, k):
      pltpu.async_copy(bufs[k], acc.at[idx_row(dst_v, i)], sems[k], add=True)

    def wait_scatter(i, k):
      pltpu.make_async_copy(bufs[k], acc.at[idx_row(dst_v, i)],
                            sems[k]).wait()

    def wait_staging(pn):
      pltpu.make_async_copy(src_hbm.at[wid, 0], src_v.at[pn], sem_i).wait()
      pltpu.make_async_copy(dst_hbm.at[wid, 0], dst_v.at[pn], sem_i).wait()

    pltpu.sync_copy(src_hbm.at[wid, 0], src_v.at[0])
    pltpu.sync_copy(dst_hbm.at[wid, 0], dst_v.at[0])
    for c in range(look):
      start_gather(c, c % ring)

    def step(i, c2):
      m = lax.rem(i, ring)
      blk = lax.div(i, sbs)
      for k in range(ring):
        @pl.when(m == k)
        def _(k=k):
          wait_gather(i, k)
          start_scatter(i, k)

          @pl.when((lax.rem(i, sbs) == 0) & (blk + 1 < nsb))
          def _():
            pn = lax.rem(blk + 1, 2)
            pltpu.async_copy(src_hbm.at[wid, blk + 1], src_v.at[pn], sem_i)
            pltpu.async_copy(dst_hbm.at[wid, blk + 1], dst_v.at[pn], sem_i)

          kn = (k + look) % ring

          @pl.when(i >= ring - look)
          def _():
            wait_scatter(i - (ring - look), kn)

          @pl.when(i + look < steps)
          def _():
            @pl.when(lax.rem(i + look, sbs) == 0)
            def _():
              wait_staging(lax.rem(lax.div(i + look, sbs), 2))

            start_gather(i + look, kn)

      return c2

    lax.fori_loop(0, steps, step, 0)
    for c in range(steps - (ring - look), steps):
      wait_scatter(c, c % ring)
    plsc.subcore_barrier()

    def writeout(rows):
      pltpu.sync_copy(acc.at[pl.ds(row0, rows)],
                      out_hbm.at[cid, pl.ds(row0, rows)])

    @pl.when(sid < _NS - 1)
    def _():
      writeout(spl)

    @pl.when(sid == _NS - 1)
    def _():
      writeout(last)

  return pl.kernel(
      body,
      out_type=jax.ShapeDtypeStruct((_NC, n, h), jnp.float32),
      mesh=plsc.VectorSubcoreMesh(core_axis_name="c", subcore_axis_name="s",
                                  num_cores=_NC, num_subcores=_NS),
      scratch_types=(
          [pltpu.VMEM_SHARED((n + _PAD_ROWS, h), jnp.float32),
           pltpu.VMEM((2, sbs, chunk), jnp.int32),
           pltpu.VMEM((2, sbs, chunk), jnp.int32)]
          + [pltpu.VMEM((chunk, h), jnp.float32)] * _RING
          + [pltpu.SemaphoreType.DMA] * (_RING + 1)
      ),
  )


def _dotf(a, b):
  return jnp.dot(a, b, preferred_element_type=jnp.float32)


def _embed_body(x_ref, w_ref, b_ref, o_ref):
  o_ref[...] = _dotf(x_ref[...], w_ref[...]) + b_ref[...]


def _gin_body(agg_ref, w1_ref, b1_ref, w2_ref, b2_ref, o_ref):
  t = agg_ref[0] + agg_ref[1]
  t = jnp.maximum(_dotf(t, w1_ref[...]) + b1_ref[...], 0.0)
  o_ref[...] = _dotf(t, w2_ref[...]) + b2_ref[...]


def _fuse_fin_body(g, n, rows, agg_ref, w1_ref, b1_ref, w2_ref, b2_ref,
                   batch_ref, ypred_ref, wf1a_ref, wf1b_ref, bf1_ref,
                   gamma_ref, beta_ref, wf2_ref, bf2_ref,
                   o_ref, z_scr, sums_scr):
  p = pl.program_id(0)
  i = pl.program_id(1)

  @pl.when(p == 0)
  def _():
    t = agg_ref[0] + agg_ref[1]
    t = jnp.maximum(_dotf(t, w1_ref[...]) + b1_ref[...], 0.0)
    h2 = _dotf(t, w2_ref[...]) + b2_ref[...]
    yproj = _dotf(ypred_ref[...], wf1b_ref[...])
    b = batch_ref[0, 0, :]
    onehot = (b[:, None] ==
              lax.broadcasted_iota(jnp.int32, (b.shape[0], g), 1)
              ).astype(jnp.float32)
    z = _dotf(h2, wf1a_ref[...]) + _dotf(onehot, yproj) + bf1_ref[...]
    z_scr[pl.ds(i * rows, rows), :] = z

    @pl.when(i == 0)
    def _():
      sums_scr[...] = jnp.zeros_like(sums_scr)

    s1 = jnp.sum(z, axis=0)
    s2 = jnp.sum(z * z, axis=0)
    sums_scr[...] = sums_scr[...] + jnp.concatenate(
        [s1[None, :], s2[None, :]], axis=0)

  @pl.when(p == 1)
  def _():
    mean = sums_scr[0, :] / n
    var = sums_scr[1, :] / n - mean * mean
    scale = lax.rsqrt(var + 1e-5) * gamma_ref[...]
    z = z_scr[pl.ds(i * rows, rows), :]
    zn = (z - mean) * scale + beta_ref[...]
    zn = jnp.maximum(zn, 0.0)
    o = _dotf(zn, wf2_ref[...]) + bf2_ref[...]
    o_ref[...] = jax.nn.sigmoid(o)


def _full(shape):
  nd = len(shape)
  return pl.BlockSpec(shape, lambda i: (0,) * nd)


def kernel(x, edge_index, batch, y_pred, W_embed, b_embed,
           W1_0, b1_0, W2_0, b2_0, W1_1, b1_1, W2_1, b2_1,
           Wf1, bf1, gamma, beta, Wf2, bf2):
  n, d = x.shape
  h = W_embed.shape[1]
  e = edge_index.shape[1]
  g, out_dim = y_pred.shape
  h2w = 2 * h
  rows = _ROWS
  n_tiles = n // rows
  assert n % rows == 0

  nw = _NC * _NS
  epw_pad = _pad_edges(e, nw)
  pad = epw_pad - e // nw
  nsb = epw_pad // (_CHUNK * _SBS)

  def shard_edges(idx, pad_val):
    shards = idx.reshape(nw, e // nw)
    if pad:
      fill = jnp.broadcast_to(pad_val, (nw, pad))
      shards = jnp.concatenate([shards, fill], axis=1)
    return shards.reshape(nw, nsb, _SBS, _CHUNK)

  pad_iota = jnp.arange(pad, dtype=jnp.int32) % _PAD_ROWS
  src = shard_edges(edge_index[0], pad_iota)
  dst = shard_edges(edge_index[1], n + pad_iota)
  zeros = jnp.zeros((n, h), jnp.float32)
  batch3 = batch.reshape(n_tiles, 1, rows)
  wf1a = Wf1[:h]
  wf1b = Wf1[h:]

  seg_sum = _make_seg_sum(n, e, h)

  row_spec = pl.BlockSpec((rows, h), lambda i: (i, 0))
  agg_spec = pl.BlockSpec((_NC, rows, h), lambda i: (0, i, 0))

  h0 = pl.pallas_call(
      _embed_body,
      grid=(n_tiles,),
      in_specs=[pl.BlockSpec((rows, d), lambda i: (i, 0)),
                _full((d, h)), _full((h,))],
      out_specs=row_spec,
      out_shape=jax.ShapeDtypeStruct((n, h), jnp.float32),
  )(x, W_embed, b_embed)

  agg0 = seg_sum(h0, src, dst, zeros)

  h1 = pl.pallas_call(
      _gin_body,
      grid=(n_tiles,),
      in_specs=[agg_spec, _full((h, h2w)), _full((h2w,)),
                _full((h2w, h)), _full((h,))],
      out_specs=row_spec,
      out_shape=jax.ShapeDtypeStruct((n, h), jnp.float32),
  )(agg0, W1_0, b1_0, W2_0, b2_0)

  agg1 = seg_sum(h1, src, dst, zeros)

  def full2(shape):
    nd = len(shape)
    return pl.BlockSpec(shape, lambda p, i: (0,) * nd)

  node_score = pl.pallas_call(
      functools.partial(_fuse_fin_body, g, float(n), rows),
      grid=(2, n_tiles),
      in_specs=[pl.BlockSpec((_NC, rows, h), lambda p, i: (0, i * (1 - p), 0)),
                full2((h, h2w)), full2((h2w,)),
                full2((h2w, h)), full2((h,)),
                pl.BlockSpec((1, 1, rows), lambda p, i: (i * (1 - p), 0, 0)),
                full2((g, out_dim)), full2((h, h2w)),
                full2((out_dim, h2w)), full2((h2w,)),
                full2((h2w,)), full2((h2w,)),
                full2((h2w, h)), full2((h,))],
      out_specs=pl.BlockSpec((rows, h), lambda p, i: (i * p, 0)),
      out_shape=jax.ShapeDtypeStruct((n, h), jnp.float32),
      scratch_shapes=[pltpu.VMEM((n, h2w), jnp.float32),
                      pltpu.VMEM((2, h2w), jnp.float32)],
  )(agg1, W1_1, b1_1, W2_1, b2_1, batch3, y_pred, wf1a, wf1b, bf1,
    gamma, beta, Wf2, bf2)

  return node_score


# speedup vs baseline: 1.0081x; 1.0081x over previous
import functools

import jax
import jax.numpy as jnp
from jax import lax
from jax.experimental import pallas as pl
from jax.experimental.pallas import tpu as pltpu
from jax.experimental.pallas import tpu_sc as plsc

_NC = 2
_NS = 16
_ROWS = 400


_CHUNK = 104
_SBS = 10
_RING = 3
_LOOK = 2
_PAD_ROWS = 8


def _pad_edges(e, nw):
  epw = e // nw
  blk = _CHUNK * _SBS
  return -(-epw // blk) * blk


def _make_seg_sum(n, e, h):
  nw = _NC * _NS
  epw = _pad_edges(e, nw)
  chunk = _CHUNK
  steps = epw // chunk
  sbs = _SBS
  assert epw % (chunk * sbs) == 0
  spl = -(-n // _NS) // 8 * 8 + 8
  last = n - (_NS - 1) * spl
  assert last > 0 and last % 8 == 0

  nsb = steps // sbs

  def body(h_hbm, src_hbm, dst_hbm, zeros_hbm, out_hbm,
           acc, src_v, dst_v, *bufs_and_sems):
    cid = lax.axis_index("c")
    sid = lax.axis_index("s")
    row0 = pl.multiple_of(sid * spl, 8)

    def seed(rows):
      @pl.when(cid == 0)
      def _():
        pltpu.sync_copy(h_hbm.at[pl.ds(row0, rows)],
                        acc.at[pl.ds(row0, rows)])

      @pl.when(cid != 0)
      def _():
        pltpu.sync_copy(zeros_hbm.at[pl.ds(row0, rows)],
                        acc.at[pl.ds(row0, rows)])

    @pl.when(sid < _NS - 1)
    def _():
      seed(spl)

    @pl.when(sid == _NS - 1)
    def _():
      seed(last)

    plsc.subcore_barrier()

    wid = sid * _NC + cid
    ring, look = _RING, _LOOK
    bufs = bufs_and_sems[:ring]
    sems = bufs_and_sems[ring:ring + ring]
    sem_i = bufs_and_sems[ring + ring]

    def idx_row(ref, i):
      return ref.at[lax.rem(lax.div(i, sbs), 2), lax.rem(i, sbs)]

    def start_gather(i, k):
      pltpu.async_copy(h_hbm.at[idx_row(src_v, i)], bufs[k], sems[k])

    def wait_gather(i, k):
      pltpu.make_async_copy(h_hbm.at[idx_row(src_v, i)], bufs[k],
                            sems[k]).wait()

    def start_scatter(i, k):
      pltpu.async_copy(bufs[k], acc.at[idx_row(dst_v, i)], sems[k], add=True)

    def wait_scatter(i, k):
      pltpu.make_async_copy(bufs[k], acc.at[idx_row(dst_v, i)],
                            sems[k]).wait()

    def wait_staging(pn):
      pltpu.make_async_copy(src_hbm.at[wid, 0], src_v.at[pn], sem_i).wait()
      pltpu.make_async_copy(dst_hbm.at[wid, 0], dst_v.at[pn], sem_i).wait()

    pltpu.sync_copy(src_hbm.at[wid, 0], src_v.at[0])
    pltpu.sync_copy(dst_hbm.at[wid, 0], dst_v.at[0])
    for c in range(look):
      start_gather(c, c % ring)

    def step(i, c2):
      m = lax.rem(i, ring)
      blk = lax.div(i, sbs)
      for k in range(ring):
        @pl.when(m == k)
        def _(k=k):
          wait_gather(i, k)
          start_scatter(i, k)

          @pl.when((lax.rem(i, sbs) == 0) & (blk + 1 < nsb))
          def _():
            pn = lax.rem(blk + 1, 2)
            pltpu.async_copy(src_hbm.at[wid, blk + 1], src_v.at[pn], sem_i)
            pltpu.async_copy(dst_hbm.at[wid, blk + 1], dst_v.at[pn], sem_i)

          kn = (k + look) % ring

          @pl.when(i >= ring - look)
          def _():
            wait_scatter(i - (ring - look), kn)

          @pl.when(i + look < steps)
          def _():
            @pl.when(lax.rem(i + look, sbs) == 0)
            def _():
              wait_staging(lax.rem(lax.div(i + look, sbs), 2))

            start_gather(i + look, kn)

      return c2

    lax.fori_loop(0, steps, step, 0)
    for c in range(steps - (ring - look), steps):
      wait_scatter(c, c % ring)
    plsc.subcore_barrier()

    def writeout(rows):
      pltpu.sync_copy(acc.at[pl.ds(row0, rows)],
                      out_hbm.at[cid, pl.ds(row0, rows)])

    @pl.when(sid < _NS - 1)
    def _():
      writeout(spl)

    @pl.when(sid == _NS - 1)
    def _():
      writeout(last)

  return pl.kernel(
      body,
      out_type=jax.ShapeDtypeStruct((_NC, n, h), jnp.float32),
      mesh=plsc.VectorSubcoreMesh(core_axis_name="c", subcore_axis_name="s",
                                  num_cores=_NC, num_subcores=_NS),
      scratch_types=(
          [pltpu.VMEM_SHARED((n + _PAD_ROWS, h), jnp.float32),
           pltpu.VMEM((2, sbs, chunk), jnp.int32),
           pltpu.VMEM((2, sbs, chunk), jnp.int32)]
          + [pltpu.VMEM((chunk, h), jnp.float32)] * _RING
          + [pltpu.SemaphoreType.DMA] * (_RING + 1)
      ),
  )


def _dotf(a, b):
  return jnp.dot(a, b, preferred_element_type=jnp.float32)


def _embed_body(x_ref, w_ref, b_ref, o_ref):
  o_ref[...] = _dotf(x_ref[...], w_ref[...]) + b_ref[...]


def _gin_body(agg_ref, w1_ref, b1_ref, w2_ref, b2_ref, o_ref):
  t = agg_ref[0] + agg_ref[1]
  t = jnp.maximum(_dotf(t, w1_ref[...]) + b1_ref[...], 0.0)
  o_ref[...] = _dotf(t, w2_ref[...]) + b2_ref[...]


def _fuse_fin_body(g, n, rows, agg_ref, w1_ref, b1_ref, w2_ref, b2_ref,
                   batch_ref, ypred_ref, wf1a_ref, wf1b_ref, bf1_ref,
                   gamma_ref, beta_ref, wf2_ref, bf2_ref,
                   o_ref, z_scr, sums_scr):
  p = pl.program_id(0)
  i = pl.program_id(1)

  @pl.when(p == 0)
  def _():
    t = agg_ref[0] + agg_ref[1]
    t = jnp.maximum(_dotf(t, w1_ref[...]) + b1_ref[...], 0.0)
    h2 = _dotf(t, w2_ref[...]) + b2_ref[...]
    yproj = _dotf(ypred_ref[...], wf1b_ref[...])
    b = batch_ref[0, 0, :]
    onehot = (b[:, None] ==
              lax.broadcasted_iota(jnp.int32, (b.shape[0], g), 1)
              ).astype(jnp.float32)
    z = _dotf(h2, wf1a_ref[...]) + _dotf(onehot, yproj) + bf1_ref[...]
    z_scr[pl.ds(i * rows, rows), :] = z

    @pl.when(i == 0)
    def _():
      sums_scr[...] = jnp.zeros_like(sums_scr)

    s1 = jnp.sum(z, axis=0)
    s2 = jnp.sum(z * z, axis=0)
    sums_scr[...] = sums_scr[...] + jnp.concatenate(
        [s1[None, :], s2[None, :]], axis=0)

  @pl.when(p == 1)
  def _():
    mean = sums_scr[0, :] / n
    var = sums_scr[1, :] / n - mean * mean
    scale = lax.rsqrt(var + 1e-5) * gamma_ref[...]
    z = z_scr[pl.ds(i * rows, rows), :]
    zn = (z - mean) * scale + beta_ref[...]
    zn = jnp.maximum(zn, 0.0)
    o = _dotf(zn, wf2_ref[...]) + bf2_ref[...]
    o_ref[...] = jax.nn.sigmoid(o)


def _full(shape):
  nd = len(shape)
  return pl.BlockSpec(shape, lambda i: (0,) * nd)


def kernel(x, edge_index, batch, y_pred, W_embed, b_embed,
           W1_0, b1_0, W2_0, b2_0, W1_1, b1_1, W2_1, b2_1,
           Wf1, bf1, gamma, beta, Wf2, bf2):
  n, d = x.shape
  h = W_embed.shape[1]
  e = edge_index.shape[1]
  g, out_dim = y_pred.shape
  h2w = 2 * h
  rows = _ROWS
  n_tiles = n // rows
  assert n % rows == 0

  nw = _NC * _NS
  epw_pad = _pad_edges(e, nw)
  pad = epw_pad - e // nw
  nsb = epw_pad // (_CHUNK * _SBS)

  def shard_edges(idx, pad_val):
    shards = idx.reshape(nw, e // nw)
    if pad:
      fill = jnp.broadcast_to(pad_val, (nw, pad))
      shards = jnp.concatenate([shards, fill], axis=1)
    return shards.reshape(nw, nsb, _SBS, _CHUNK)

  pad_iota = jnp.arange(pad, dtype=jnp.int32) % _PAD_ROWS
  src = shard_edges(edge_index[0], pad_iota)
  dst = shard_edges(edge_index[1], n + pad_iota)
  zeros = jnp.zeros((n, h), jnp.float32)
  batch3 = batch.reshape(n_tiles, 1, rows)
  wf1a = Wf1[:h]
  wf1b = Wf1[h:]

  seg_sum = _make_seg_sum(n, e, h)

  row_spec = pl.BlockSpec((rows, h), lambda i: (i, 0))
  agg_spec = pl.BlockSpec((_NC, rows, h), lambda i: (0, i, 0))

  h0 = pl.pallas_call(
      _embed_body,
      grid=(n_tiles,),
      in_specs=[pl.BlockSpec((rows, d), lambda i: (i, 0)),
                _full((d, h)), _full((h,))],
      out_specs=row_spec,
      out_shape=jax.ShapeDtypeStruct((n, h), jnp.float32),
  )(x, W_embed, b_embed)

  agg0 = seg_sum(h0, src, dst, zeros)

  h1 = pl.pallas_call(
      _gin_body,
      grid=(n_tiles,),
      in_specs=[agg_spec, _full((h, h2w)), _full((h2w,)),
                _full((h2w, h)), _full((h,))],
      out_specs=row_spec,
      out_shape=jax.ShapeDtypeStruct((n, h), jnp.float32),
  )(agg0, W1_0, b1_0, W2_0, b2_0)

  agg1 = seg_sum(h1, src, dst, zeros)

  def full2(shape):
    nd = len(shape)
    return pl.BlockSpec(shape, lambda p, i: (0,) * nd)

  node_score = pl.pallas_call(
      functools.partial(_fuse_fin_body, g, float(n), rows),
      grid=(2, n_tiles),
      in_specs=[pl.BlockSpec((_NC, rows, h), lambda p, i: (0, i * (1 - p), 0)),
                full2((h, h2w)), full2((h2w,)),
                full2((h2w, h)), full2((h,)),
                pl.BlockSpec((1, 1, rows), lambda p, i: (i * (1 - p), 0, 0)),
                full2((g, out_dim)), full2((h, h2w)),
                full2((out_dim, h2w)), full2((h2w,)),
                full2((h2w,)), full2((h2w,)),
                full2((h2w, h)), full2((h,))],
      out_specs=pl.BlockSpec((rows, h), lambda p, i: (i * p, 0)),
      out_shape=jax.ShapeDtypeStruct((n, h), jnp.float32),
      scratch_shapes=[pltpu.VMEM((n, h2w), jnp.float32),
                      pltpu.VMEM((2, h2w), jnp.float32)],
  )(agg1, W1_1, b1_1, W2_1, b2_1, batch3, y_pred, wf1a, wf1b, bf1,
    gamma, beta, Wf2, bf2)

  return node_score


# speedup vs baseline: 1.3136x; 1.3030x over previous
import functools

import jax
import jax.numpy as jnp
from jax import lax
from jax.experimental import pallas as pl
from jax.experimental.pallas import tpu as pltpu
from jax.experimental.pallas import tpu_sc as plsc

_NC = 2
_NS = 16
_ROWS = 400


_CHUNK = 96
_SBS = 21
_RING = 3
_LOOK = 2
_PAD_ROWS = 8


def _pad_edges(e, nw):
  epw = e // nw
  blk = _CHUNK * _SBS
  return -(-epw // blk) * blk


def _make_seg_sum(n, e, h):
  nw = _NC * _NS
  epw = _pad_edges(e, nw)
  chunk = _CHUNK
  steps = epw // chunk
  sbs = _SBS
  assert epw % (chunk * sbs) == 0
  spl = -(-n // _NS) // 8 * 8 + 8
  last = n - (_NS - 1) * spl
  assert last > 0 and last % 8 == 0

  nsb = steps // sbs

  def body(h_hbm, src_hbm, dst_hbm, zeros_hbm, out_hbm,
           acc, src_v, dst_v, *bufs_and_sems):
    cid = lax.axis_index("c")
    sid = lax.axis_index("s")
    row0 = pl.multiple_of(sid * spl, 8)

    def seed(rows):
      @pl.when(cid == 0)
      def _():
        pltpu.sync_copy(h_hbm.at[pl.ds(row0, rows)],
                        acc.at[pl.ds(row0, rows)])

      @pl.when(cid != 0)
      def _():
        pltpu.sync_copy(zeros_hbm.at[pl.ds(row0, rows)],
                        acc.at[pl.ds(row0, rows)])

    @pl.when(sid < _NS - 1)
    def _():
      seed(spl)

    @pl.when(sid == _NS - 1)
    def _():
      seed(last)

    plsc.subcore_barrier()

    wid = sid * _NC + cid
    ring, look = _RING, _LOOK
    bufs = bufs_and_sems[:ring]
    sems = bufs_and_sems[ring:ring + ring]
    sem_i = bufs_and_sems[ring + ring]

    def idx_row(ref, i):
      return ref.at[lax.rem(lax.div(i, sbs), 2), lax.rem(i, sbs)]

    def start_gather(i, k):
      pltpu.async_copy(h_hbm.at[idx_row(src_v, i)], bufs[k], sems[k])

    def wait_gather(i, k):
      pltpu.make_async_copy(h_hbm.at[idx_row(src_v, i)], bufs[k],
                            sems[k]).wait()

    def start_scatter(i, k):
      pltpu.async_copy(bufs[k], acc.at[idx_row(dst_v, i)], sems[k], add=True)

    def wait_scatter(i, k):
      pltpu.make_async_copy(bufs[k], acc.at[idx_row(dst_v, i)],
                            sems[k]).wait()

    def wait_staging(pn):
      pltpu.make_async_copy(src_hbm.at[wid, 0], src_v.at[pn], sem_i).wait()
      pltpu.make_async_copy(dst_hbm.at[wid, 0], dst_v.at[pn], sem_i).wait()

    pltpu.sync_copy(src_hbm.at[wid, 0], src_v.at[0])
    pltpu.sync_copy(dst_hbm.at[wid, 0], dst_v.at[0])
    for c in range(look):
      start_gather(c, c % ring)

    def step(i, c2):
      m = lax.rem(i, ring)
      blk = lax.div(i, sbs)
      for k in range(ring):
        @pl.when(m == k)
        def _(k=k):
          wait_gather(i, k)
          start_scatter(i, k)

          @pl.when((lax.rem(i, sbs) == 0) & (blk + 1 < nsb))
          def _():
            pn = lax.rem(blk + 1, 2)
            pltpu.async_copy(src_hbm.at[wid, blk + 1], src_v.at[pn], sem_i)
            pltpu.async_copy(dst_hbm.at[wid, blk + 1], dst_v.at[pn], sem_i)

          kn = (k + look) % ring

          @pl.when(i >= ring - look)
          def _():
            wait_scatter(i - (ring - look), kn)

          @pl.when(i + look < steps)
          def _():
            @pl.when(lax.rem(i + look, sbs) == 0)
            def _():
              wait_staging(lax.rem(lax.div(i + look, sbs), 2))

            start_gather(i + look, kn)

      return c2

    lax.fori_loop(0, steps, step, 0)
    for c in range(steps - (ring - look), steps):
      wait_scatter(c, c % ring)
    plsc.subcore_barrier()

    def writeout(rows):
      pltpu.sync_copy(acc.at[pl.ds(row0, rows)],
                      out_hbm.at[cid, pl.ds(row0, rows)])

    @pl.when(sid < _NS - 1)
    def _():
      writeout(spl)

    @pl.when(sid == _NS - 1)
    def _():
      writeout(last)

  return pl.kernel(
      body,
      out_type=jax.ShapeDtypeStruct((_NC, n, h), jnp.float32),
      mesh=plsc.VectorSubcoreMesh(core_axis_name="c", subcore_axis_name="s",
                                  num_cores=_NC, num_subcores=_NS),
      scratch_types=(
          [pltpu.VMEM_SHARED((n + _PAD_ROWS, h), jnp.float32),
           pltpu.VMEM((2, sbs, chunk), jnp.int32),
           pltpu.VMEM((2, sbs, chunk), jnp.int32)]
          + [pltpu.VMEM((chunk, h), jnp.float32)] * _RING
          + [pltpu.SemaphoreType.DMA] * (_RING + 1)
      ),
  )


def _dotf(a, b):
  return jnp.dot(a, b, preferred_element_type=jnp.float32)


def _embed_body(x_ref, w_ref, b_ref, o_ref):
  o_ref[...] = _dotf(x_ref[...], w_ref[...]) + b_ref[...]


def _gin_body(agg_ref, w1_ref, b1_ref, w2_ref, b2_ref, o_ref):
  t = agg_ref[0] + agg_ref[1]
  t = jnp.maximum(_dotf(t, w1_ref[...]) + b1_ref[...], 0.0)
  o_ref[...] = _dotf(t, w2_ref[...]) + b2_ref[...]


def _fuse_fin_body(g, n, rows, agg_ref, w1_ref, b1_ref, w2_ref, b2_ref,
                   batch_ref, ypred_ref, wf1a_ref, wf1b_ref, bf1_ref,
                   gamma_ref, beta_ref, wf2_ref, bf2_ref,
                   o_ref, z_scr, sums_scr):
  p = pl.program_id(0)
  i = pl.program_id(1)

  @pl.when(p == 0)
  def _():
    t = agg_ref[0] + agg_ref[1]
    t = jnp.maximum(_dotf(t, w1_ref[...]) + b1_ref[...], 0.0)
    h2 = _dotf(t, w2_ref[...]) + b2_ref[...]
    yproj = _dotf(ypred_ref[...], wf1b_ref[...])
    b = batch_ref[0, 0, :]
    onehot = (b[:, None] ==
              lax.broadcasted_iota(jnp.int32, (b.shape[0], g), 1)
              ).astype(jnp.float32)
    z = _dotf(h2, wf1a_ref[...]) + _dotf(onehot, yproj) + bf1_ref[...]
    z_scr[pl.ds(i * rows, rows), :] = z

    @pl.when(i == 0)
    def _():
      sums_scr[...] = jnp.zeros_like(sums_scr)

    s1 = jnp.sum(z, axis=0)
    s2 = jnp.sum(z * z, axis=0)
    sums_scr[...] = sums_scr[...] + jnp.concatenate(
        [s1[None, :], s2[None, :]], axis=0)

  @pl.when(p == 1)
  def _():
    mean = sums_scr[0, :] / n
    var = sums_scr[1, :] / n - mean * mean
    scale = lax.rsqrt(var + 1e-5) * gamma_ref[...]
    z = z_scr[pl.ds(i * rows, rows), :]
    zn = (z - mean) * scale + beta_ref[...]
    zn = jnp.maximum(zn, 0.0)
    o = _dotf(zn, wf2_ref[...]) + bf2_ref[...]
    o_ref[...] = jax.nn.sigmoid(o)


def _full(shape):
  nd = len(shape)
  return pl.BlockSpec(shape, lambda i: (0,) * nd)


def kernel(x, edge_index, batch, y_pred, W_embed, b_embed,
           W1_0, b1_0, W2_0, b2_0, W1_1, b1_1, W2_1, b2_1,
           Wf1, bf1, gamma, beta, Wf2, bf2):
  n, d = x.shape
  h = W_embed.shape[1]
  e = edge_index.shape[1]
  g, out_dim = y_pred.shape
  h2w = 2 * h
  rows = _ROWS
  n_tiles = n // rows
  assert n % rows == 0

  nw = _NC * _NS
  epw_pad = _pad_edges(e, nw)
  pad = epw_pad - e // nw
  nsb = epw_pad // (_CHUNK * _SBS)

  def shard_edges(idx, pad_val):
    shards = idx.reshape(nw, e // nw)
    if pad:
      fill = jnp.broadcast_to(pad_val, (nw, pad))
      shards = jnp.concatenate([shards, fill], axis=1)
    return shards.reshape(nw, nsb, _SBS, _CHUNK)

  pad_iota = jnp.arange(pad, dtype=jnp.int32) % _PAD_ROWS
  src = shard_edges(edge_index[0], pad_iota)
  dst = shard_edges(edge_index[1], n + pad_iota)
  zeros = jnp.zeros((n, h), jnp.float32)
  batch3 = batch.reshape(n_tiles, 1, rows)
  wf1a = Wf1[:h]
  wf1b = Wf1[h:]

  seg_sum = _make_seg_sum(n, e, h)

  row_spec = pl.BlockSpec((rows, h), lambda i: (i, 0))
  agg_spec = pl.BlockSpec((_NC, rows, h), lambda i: (0, i, 0))

  h0 = pl.pallas_call(
      _embed_body,
      grid=(n_tiles,),
      in_specs=[pl.BlockSpec((rows, d), lambda i: (i, 0)),
                _full((d, h)), _full((h,))],
      out_specs=row_spec,
      out_shape=jax.ShapeDtypeStruct((n, h), jnp.float32),
  )(x, W_embed, b_embed)

  agg0 = seg_sum(h0, src, dst, zeros)

  h1 = pl.pallas_call(
      _gin_body,
      grid=(n_tiles,),
      in_specs=[agg_spec, _full((h, h2w)), _full((h2w,)),
                _full((h2w, h)), _full((h,))],
      out_specs=row_spec,
      out_shape=jax.ShapeDtypeStruct((n, h), jnp.float32),
  )(agg0, W1_0, b1_0, W2_0, b2_0)

  agg1 = seg_sum(h1, src, dst, zeros)

  def full2(shape):
    nd = len(shape)
    return pl.BlockSpec(shape, lambda p, i: (0,) * nd)

  node_score = pl.pallas_call(
      functools.partial(_fuse_fin_body, g, float(n), rows),
      grid=(2, n_tiles),
      in_specs=[pl.BlockSpec((_NC, rows, h), lambda p, i: (0, i * (1 - p), 0)),
                full2((h, h2w)), full2((h2w,)),
                full2((h2w, h)), full2((h,)),
                pl.BlockSpec((1, 1, rows), lambda p, i: (i * (1 - p), 0, 0)),
                full2((g, out_dim)), full2((h, h2w)),
                full2((out_dim, h2w)), full2((h2w,)),
                full2((h2w,)), full2((h2w,)),
                full2((h2w, h)), full2((h,))],
      out_specs=pl.BlockSpec((rows, h), lambda p, i: (i * p, 0)),
      out_shape=jax.ShapeDtypeStruct((n, h), jnp.float32),
      scratch_shapes=[pltpu.VMEM((n, h2w), jnp.float32),
                      pltpu.VMEM((2, h2w), jnp.float32)],
  )(agg1, W1_1, b1_1, W2_1, b2_1, batch3, y_pred, wf1a, wf1b, bf1,
    gamma, beta, Wf2, bf2)

  return node_score


# speedup vs baseline: 1.5431x; 1.1747x over previous
import functools

import jax
import jax.numpy as jnp
from jax import lax
from jax.experimental import pallas as pl
from jax.experimental.pallas import tpu as pltpu
from jax.experimental.pallas import tpu_sc as plsc

_NC = 2
_NS = 16
_ROWS = 1000


_CHUNK = 80
_SBS = 25
_RING = 3
_LOOK = 2
_PAD_ROWS = 8


def _pad_edges(e, nw):
  epw = e // nw
  blk = _CHUNK * _SBS
  return -(-epw // blk) * blk


def _make_seg_sum(n, e, h):
  nw = _NC * _NS
  epw = _pad_edges(e, nw)
  chunk = _CHUNK
  steps = epw // chunk
  sbs = _SBS
  assert epw % (chunk * sbs) == 0
  spl = -(-n // _NS) // 8 * 8 + 8
  last = n - (_NS - 1) * spl
  assert last > 0 and last % 8 == 0

  nsb = steps // sbs

  def body(h_hbm, src_hbm, dst_hbm, zeros_hbm, out_hbm,
           acc, src_v, dst_v, *bufs_and_sems):
    cid = lax.axis_index("c")
    sid = lax.axis_index("s")
    row0 = pl.multiple_of(sid * spl, 8)

    def seed(rows):
      @pl.when(cid == 0)
      def _():
        pltpu.sync_copy(h_hbm.at[pl.ds(row0, rows)],
                        acc.at[pl.ds(row0, rows)])

      @pl.when(cid != 0)
      def _():
        pltpu.sync_copy(zeros_hbm.at[pl.ds(row0, rows)],
                        acc.at[pl.ds(row0, rows)])

    @pl.when(sid < _NS - 1)
    def _():
      seed(spl)

    @pl.when(sid == _NS - 1)
    def _():
      seed(last)

    plsc.subcore_barrier()

    wid = sid * _NC + cid
    ring, look = _RING, _LOOK
    bufs = bufs_and_sems[:ring]
    sems = bufs_and_sems[ring:ring + ring]
    sem_i = bufs_and_sems[ring + ring]

    def idx_row(ref, i):
      return ref.at[lax.rem(lax.div(i, sbs), 2), lax.rem(i, sbs)]

    def start_gather(i, k):
      pltpu.async_copy(h_hbm.at[idx_row(src_v, i)], bufs[k], sems[k])

    def wait_gather(i, k):
      pltpu.make_async_copy(h_hbm.at[idx_row(src_v, i)], bufs[k],
                            sems[k]).wait()

    def start_scatter(i, k):
      pltpu.async_copy(bufs[k], acc.at[idx_row(dst_v, i)], sems[k], add=True)

    def wait_scatter(i, k):
      pltpu.make_async_copy(bufs[k], acc.at[idx_row(dst_v, i)],
                            sems[k]).wait()

    def wait_staging(pn):
      pltpu.make_async_copy(src_hbm.at[wid, 0], src_v.at[pn], sem_i).wait()
      pltpu.make_async_copy(dst_hbm.at[wid, 0], dst_v.at[pn], sem_i).wait()

    pltpu.sync_copy(src_hbm.at[wid, 0], src_v.at[0])
    pltpu.sync_copy(dst_hbm.at[wid, 0], dst_v.at[0])
    for c in range(look):
      start_gather(c, c % ring)

    def step(i, c2):
      m = lax.rem(i, ring)
      blk = lax.div(i, sbs)
      for k in range(ring):
        @pl.when(m == k)
        def _(k=k):
          wait_gather(i, k)
          start_scatter(i, k)

          @pl.when((lax.rem(i, sbs) == 0) & (blk + 1 < nsb))
          def _():
            pn = lax.rem(blk + 1, 2)
            pltpu.async_copy(src_hbm.at[wid, blk + 1], src_v.at[pn], sem_i)
            pltpu.async_copy(dst_hbm.at[wid, blk + 1], dst_v.at[pn], sem_i)

          kn = (k + look) % ring

          @pl.when(i >= ring - look)
          def _():
            wait_scatter(i - (ring - look), kn)

          @pl.when(i + look < steps)
          def _():
            @pl.when(lax.rem(i + look, sbs) == 0)
            def _():
              wait_staging(lax.rem(lax.div(i + look, sbs), 2))

            start_gather(i + look, kn)

      return c2

    lax.fori_loop(0, steps, step, 0)
    for c in range(steps - (ring - look), steps):
      wait_scatter(c, c % ring)
    plsc.subcore_barrier()

    def writeout(rows):
      pltpu.sync_copy(acc.at[pl.ds(row0, rows)],
                      out_hbm.at[cid, pl.ds(row0, rows)])

    @pl.when(sid < _NS - 1)
    def _():
      writeout(spl)

    @pl.when(sid == _NS - 1)
    def _():
      writeout(last)

  return pl.kernel(
      body,
      out_type=jax.ShapeDtypeStruct((_NC, n, h), jnp.float32),
      mesh=plsc.VectorSubcoreMesh(core_axis_name="c", subcore_axis_name="s",
                                  num_cores=_NC, num_subcores=_NS),
      scratch_types=(
          [pltpu.VMEM_SHARED((n + _PAD_ROWS, h), jnp.float32),
           pltpu.VMEM((2, sbs, chunk), jnp.int32),
           pltpu.VMEM((2, sbs, chunk), jnp.int32)]
          + [pltpu.VMEM((chunk, h), jnp.float32)] * _RING
          + [pltpu.SemaphoreType.DMA] * (_RING + 1)
      ),
  )


def _dotf(a, b):
  return jnp.dot(a, b, preferred_element_type=jnp.float32)


def _embed_body(x_ref, w_ref, b_ref, o_ref):
  o_ref[...] = _dotf(x_ref[...], w_ref[...]) + b_ref[...]


def _gin_body(agg_ref, w1_ref, b1_ref, w2_ref, b2_ref, o_ref):
  t = agg_ref[0] + agg_ref[1]
  t = jnp.maximum(_dotf(t, w1_ref[...]) + b1_ref[...], 0.0)
  o_ref[...] = _dotf(t, w2_ref[...]) + b2_ref[...]


def _fuse_fin_body(g, n, rows, agg_ref, w1_ref, b1_ref, w2_ref, b2_ref,
                   batch_ref, ypred_ref, wf1a_ref, wf1b_ref, bf1_ref,
                   gamma_ref, beta_ref, wf2_ref, bf2_ref,
                   o_ref, z_scr, sums_scr):
  p = pl.program_id(0)
  i = pl.program_id(1)

  @pl.when(p == 0)
  def _():
    t = agg_ref[0] + agg_ref[1]
    t = jnp.maximum(_dotf(t, w1_ref[...]) + b1_ref[...], 0.0)
    h2 = _dotf(t, w2_ref[...]) + b2_ref[...]
    yproj = _dotf(ypred_ref[...], wf1b_ref[...])
    b = batch_ref[0, 0, :]
    onehot = (b[:, None] ==
              lax.broadcasted_iota(jnp.int32, (b.shape[0], g), 1)
              ).astype(jnp.float32)
    z = _dotf(h2, wf1a_ref[...]) + _dotf(onehot, yproj) + bf1_ref[...]
    z_scr[pl.ds(i * rows, rows), :] = z

    @pl.when(i == 0)
    def _():
      sums_scr[...] = jnp.zeros_like(sums_scr)

    s1 = jnp.sum(z, axis=0)
    s2 = jnp.sum(z * z, axis=0)
    sums_scr[...] = sums_scr[...] + jnp.concatenate(
        [s1[None, :], s2[None, :]], axis=0)

  @pl.when(p == 1)
  def _():
    mean = sums_scr[0, :] / n
    var = sums_scr[1, :] / n - mean * mean
    scale = lax.rsqrt(var + 1e-5) * gamma_ref[...]
    z = z_scr[pl.ds(i * rows, rows), :]
    zn = (z - mean) * scale + beta_ref[...]
    zn = jnp.maximum(zn, 0.0)
    o = _dotf(zn, wf2_ref[...]) + bf2_ref[...]
    o_ref[...] = jax.nn.sigmoid(o)


def _full(shape):
  nd = len(shape)
  return pl.BlockSpec(shape, lambda i: (0,) * nd)


def kernel(x, edge_index, batch, y_pred, W_embed, b_embed,
           W1_0, b1_0, W2_0, b2_0, W1_1, b1_1, W2_1, b2_1,
           Wf1, bf1, gamma, beta, Wf2, bf2):
  n, d = x.shape
  h = W_embed.shape[1]
  e = edge_index.shape[1]
  g, out_dim = y_pred.shape
  h2w = 2 * h
  rows = _ROWS
  n_tiles = n // rows
  assert n % rows == 0

  nw = _NC * _NS
  epw_pad = _pad_edges(e, nw)
  pad = epw_pad - e // nw
  nsb = epw_pad // (_CHUNK * _SBS)

  def shard_edges(idx, pad_val):
    shards = idx.reshape(nw, e // nw)
    if pad:
      fill = jnp.broadcast_to(pad_val, (nw, pad))
      shards = jnp.concatenate([shards, fill], axis=1)
    return shards.reshape(nw, nsb, _SBS, _CHUNK)

  pad_iota = jnp.arange(pad, dtype=jnp.int32) % _PAD_ROWS
  src = shard_edges(edge_index[0], pad_iota)
  dst = shard_edges(edge_index[1], n + pad_iota)
  zeros = jnp.zeros((n, h), jnp.float32)
  batch3 = batch.reshape(n_tiles, 1, rows)
  wf1a = Wf1[:h]
  wf1b = Wf1[h:]

  seg_sum = _make_seg_sum(n, e, h)

  row_spec = pl.BlockSpec((rows, h), lambda i: (i, 0))
  agg_spec = pl.BlockSpec((_NC, rows, h), lambda i: (0, i, 0))

  h0 = pl.pallas_call(
      _embed_body,
      grid=(n_tiles,),
      in_specs=[pl.BlockSpec((rows, d), lambda i: (i, 0)),
                _full((d, h)), _full((h,))],
      out_specs=row_spec,
      out_shape=jax.ShapeDtypeStruct((n, h), jnp.float32),
  )(x, W_embed, b_embed)

  agg0 = seg_sum(h0, src, dst, zeros)

  h1 = pl.pallas_call(
      _gin_body,
      grid=(n_tiles,),
      in_specs=[agg_spec, _full((h, h2w)), _full((h2w,)),
                _full((h2w, h)), _full((h,))],
      out_specs=row_spec,
      out_shape=jax.ShapeDtypeStruct((n, h), jnp.float32),
  )(agg0, W1_0, b1_0, W2_0, b2_0)

  agg1 = seg_sum(h1, src, dst, zeros)

  def full2(shape):
    nd = len(shape)
    return pl.BlockSpec(shape, lambda p, i: (0,) * nd)

  node_score = pl.pallas_call(
      functools.partial(_fuse_fin_body, g, float(n), rows),
      grid=(2, n_tiles),
      in_specs=[pl.BlockSpec((_NC, rows, h), lambda p, i: (0, i * (1 - p), 0)),
                full2((h, h2w)), full2((h2w,)),
                full2((h2w, h)), full2((h,)),
                pl.BlockSpec((1, 1, rows), lambda p, i: (i * (1 - p), 0, 0)),
                full2((g, out_dim)), full2((h, h2w)),
                full2((out_dim, h2w)), full2((h2w,)),
                full2((h2w,)), full2((h2w,)),
                full2((h2w, h)), full2((h,))],
      out_specs=pl.BlockSpec((rows, h), lambda p, i: (i * p, 0)),
      out_shape=jax.ShapeDtypeStruct((n, h), jnp.float32),
      scratch_shapes=[pltpu.VMEM((n, h2w), jnp.float32),
                      pltpu.VMEM((2, h2w), jnp.float32)],
  )(agg1, W1_1, b1_1, W2_1, b2_1, batch3, y_pred, wf1a, wf1b, bf1,
    gamma, beta, Wf2, bf2)

  return node_score


# speedup vs baseline: 1.5916x; 1.0314x over previous
import functools

import jax
import jax.numpy as jnp
from jax import lax
from jax.experimental import pallas as pl
from jax.experimental.pallas import tpu as pltpu
from jax.experimental.pallas import tpu_sc as plsc

_NC = 2
_NS = 16
_ROWS = 2000


_CHUNK = 80
_SBS = 25
_RING = 3
_LOOK = 2
_PAD_ROWS = 8


def _pad_edges(e, nw):
  epw = e // nw
  blk = _CHUNK * _SBS
  return -(-epw // blk) * blk


def _make_seg_sum(n, e, h):
  nw = _NC * _NS
  epw = _pad_edges(e, nw)
  chunk = _CHUNK
  steps = epw // chunk
  sbs = _SBS
  assert epw % (chunk * sbs) == 0
  spl = -(-n // _NS) // 8 * 8 + 8
  last = n - (_NS - 1) * spl
  assert last > 0 and last % 8 == 0

  nsb = steps // sbs

  def body(h_hbm, src_hbm, dst_hbm, zeros_hbm, out_hbm,
           acc, src_v, dst_v, *bufs_and_sems):
    cid = lax.axis_index("c")
    sid = lax.axis_index("s")
    row0 = pl.multiple_of(sid * spl, 8)

    def seed(rows):
      @pl.when(cid == 0)
      def _():
        pltpu.sync_copy(h_hbm.at[pl.ds(row0, rows)],
                        acc.at[pl.ds(row0, rows)])

      @pl.when(cid != 0)
      def _():
        pltpu.sync_copy(zeros_hbm.at[pl.ds(row0, rows)],
                        acc.at[pl.ds(row0, rows)])

    @pl.when(sid < _NS - 1)
    def _():
      seed(spl)

    @pl.when(sid == _NS - 1)
    def _():
      seed(last)

    plsc.subcore_barrier()

    wid = sid * _NC + cid
    ring, look = _RING, _LOOK
    bufs = bufs_and_sems[:ring]
    sems = bufs_and_sems[ring:ring + ring]
    sem_i = bufs_and_sems[ring + ring]

    def idx_row(ref, i):
      return ref.at[lax.rem(lax.div(i, sbs), 2), lax.rem(i, sbs)]

    def start_gather(i, k):
      pltpu.async_copy(h_hbm.at[idx_row(src_v, i)], bufs[k], sems[k])

    def wait_gather(i, k):
      pltpu.make_async_copy(h_hbm.at[idx_row(src_v, i)], bufs[k],
                            sems[k]).wait()

    def start_scatter(i, k):
      pltpu.async_copy(bufs[k], acc.at[idx_row(dst_v, i)], sems[k], add=True)

    def wait_scatter(i, k):
      pltpu.make_async_copy(bufs[k], acc.at[idx_row(dst_v, i)],
                            sems[k]).wait()

    def wait_staging(pn):
      pltpu.make_async_copy(src_hbm.at[wid, 0], src_v.at[pn], sem_i).wait()
      pltpu.make_async_copy(dst_hbm.at[wid, 0], dst_v.at[pn], sem_i).wait()

    pltpu.sync_copy(src_hbm.at[wid, 0], src_v.at[0])
    pltpu.sync_copy(dst_hbm.at[wid, 0], dst_v.at[0])
    for c in range(look):
      start_gather(c, c % ring)

    def step(i, c2):
      m = lax.rem(i, ring)
      blk = lax.div(i, sbs)
      for k in range(ring):
        @pl.when(m == k)
        def _(k=k):
          wait_gather(i, k)
          start_scatter(i, k)

          @pl.when((lax.rem(i, sbs) == 0) & (blk + 1 < nsb))
          def _():
            pn = lax.rem(blk + 1, 2)
            pltpu.async_copy(src_hbm.at[wid, blk + 1], src_v.at[pn], sem_i)
            pltpu.async_copy(dst_hbm.at[wid, blk + 1], dst_v.at[pn], sem_i)

          kn = (k + look) % ring

          @pl.when(i >= ring - look)
          def _():
            wait_scatter(i - (ring - look), kn)

          @pl.when(i + look < steps)
          def _():
            @pl.when(lax.rem(i + look, sbs) == 0)
            def _():
              wait_staging(lax.rem(lax.div(i + look, sbs), 2))

            start_gather(i + look, kn)

      return c2

    lax.fori_loop(0, steps, step, 0)
    for c in range(steps - (ring - look), steps):
      wait_scatter(c, c % ring)
    plsc.subcore_barrier()

    def writeout(rows):
      pltpu.sync_copy(acc.at[pl.ds(row0, rows)],
                      out_hbm.at[cid, pl.ds(row0, rows)])

    @pl.when(sid < _NS - 1)
    def _():
      writeout(spl)

    @pl.when(sid == _NS - 1)
    def _():
      writeout(last)

  return pl.kernel(
      body,
      out_type=jax.ShapeDtypeStruct((_NC, n, h), jnp.float32),
      mesh=plsc.VectorSubcoreMesh(core_axis_name="c", subcore_axis_name="s",
                                  num_cores=_NC, num_subcores=_NS),
      scratch_types=(
          [pltpu.VMEM_SHARED((n + _PAD_ROWS, h), jnp.float32),
           pltpu.VMEM((2, sbs, chunk), jnp.int32),
           pltpu.VMEM((2, sbs, chunk), jnp.int32)]
          + [pltpu.VMEM((chunk, h), jnp.float32)] * _RING
          + [pltpu.SemaphoreType.DMA] * (_RING + 1)
      ),
  )


def _dotf(a, b):
  return jnp.dot(a, b, preferred_element_type=jnp.float32)


def _embed_body(x_ref, w_ref, b_ref, o_ref):
  o_ref[...] = _dotf(x_ref[...], w_ref[...]) + b_ref[...]


def _gin_body(agg_ref, w1_ref, b1_ref, w2_ref, b2_ref, o_ref):
  t = agg_ref[0] + agg_ref[1]
  t = jnp.maximum(_dotf(t, w1_ref[...]) + b1_ref[...], 0.0)
  o_ref[...] = _dotf(t, w2_ref[...]) + b2_ref[...]


def _fuse_fin_body(g, n, rows, agg_ref, w1_ref, b1_ref, w2_ref, b2_ref,
                   batch_ref, ypred_ref, wf1a_ref, wf1b_ref, bf1_ref,
                   gamma_ref, beta_ref, wf2_ref, bf2_ref,
                   o_ref, z_scr, sums_scr):
  p = pl.program_id(0)
  i = pl.program_id(1)

  @pl.when(p == 0)
  def _():
    t = agg_ref[0] + agg_ref[1]
    t = jnp.maximum(_dotf(t, w1_ref[...]) + b1_ref[...], 0.0)
    h2 = _dotf(t, w2_ref[...]) + b2_ref[...]
    yproj = _dotf(ypred_ref[...], wf1b_ref[...])
    b = batch_ref[0, 0, :]
    onehot = (b[:, None] ==
              lax.broadcasted_iota(jnp.int32, (b.shape[0], g), 1)
              ).astype(jnp.float32)
    z = _dotf(h2, wf1a_ref[...]) + _dotf(onehot, yproj) + bf1_ref[...]
    z_scr[pl.ds(i * rows, rows), :] = z

    @pl.when(i == 0)
    def _():
      sums_scr[...] = jnp.zeros_like(sums_scr)

    s1 = jnp.sum(z, axis=0)
    s2 = jnp.sum(z * z, axis=0)
    sums_scr[...] = sums_scr[...] + jnp.concatenate(
        [s1[None, :], s2[None, :]], axis=0)

  @pl.when(p == 1)
  def _():
    mean = sums_scr[0, :] / n
    var = sums_scr[1, :] / n - mean * mean
    scale = lax.rsqrt(var + 1e-5) * gamma_ref[...]
    z = z_scr[pl.ds(i * rows, rows), :]
    zn = (z - mean) * scale + beta_ref[...]
    zn = jnp.maximum(zn, 0.0)
    o = _dotf(zn, wf2_ref[...]) + bf2_ref[...]
    o_ref[...] = jax.nn.sigmoid(o)


def _full(shape):
  nd = len(shape)
  return pl.BlockSpec(shape, lambda i: (0,) * nd)


def kernel(x, edge_index, batch, y_pred, W_embed, b_embed,
           W1_0, b1_0, W2_0, b2_0, W1_1, b1_1, W2_1, b2_1,
           Wf1, bf1, gamma, beta, Wf2, bf2):
  n, d = x.shape
  h = W_embed.shape[1]
  e = edge_index.shape[1]
  g, out_dim = y_pred.shape
  h2w = 2 * h
  rows = _ROWS
  n_tiles = n // rows
  assert n % rows == 0

  nw = _NC * _NS
  epw_pad = _pad_edges(e, nw)
  pad = epw_pad - e // nw
  nsb = epw_pad // (_CHUNK * _SBS)

  def shard_edges(idx, pad_val):
    shards = idx.reshape(nw, e // nw)
    if pad:
      fill = jnp.broadcast_to(pad_val, (nw, pad))
      shards = jnp.concatenate([shards, fill], axis=1)
    return shards.reshape(nw, nsb, _SBS, _CHUNK)

  pad_iota = jnp.arange(pad, dtype=jnp.int32) % _PAD_ROWS
  src = shard_edges(edge_index[0], pad_iota)
  dst = shard_edges(edge_index[1], n + pad_iota)
  zeros = jnp.zeros((n, h), jnp.float32)
  batch3 = batch.reshape(n_tiles, 1, rows)
  wf1a = Wf1[:h]
  wf1b = Wf1[h:]

  seg_sum = _make_seg_sum(n, e, h)

  row_spec = pl.BlockSpec((rows, h), lambda i: (i, 0))
  agg_spec = pl.BlockSpec((_NC, rows, h), lambda i: (0, i, 0))

  h0 = pl.pallas_call(
      _embed_body,
      grid=(n_tiles,),
      in_specs=[pl.BlockSpec((rows, d), lambda i: (i, 0)),
                _full((d, h)), _full((h,))],
      out_specs=row_spec,
      out_shape=jax.ShapeDtypeStruct((n, h), jnp.float32),
  )(x, W_embed, b_embed)

  agg0 = seg_sum(h0, src, dst, zeros)

  h1 = pl.pallas_call(
      _gin_body,
      grid=(n_tiles,),
      in_specs=[agg_spec, _full((h, h2w)), _full((h2w,)),
                _full((h2w, h)), _full((h,))],
      out_specs=row_spec,
      out_shape=jax.ShapeDtypeStruct((n, h), jnp.float32),
  )(agg0, W1_0, b1_0, W2_0, b2_0)

  agg1 = seg_sum(h1, src, dst, zeros)

  def full2(shape):
    nd = len(shape)
    return pl.BlockSpec(shape, lambda p, i: (0,) * nd)

  node_score = pl.pallas_call(
      functools.partial(_fuse_fin_body, g, float(n), rows),
      grid=(2, n_tiles),
      in_specs=[pl.BlockSpec((_NC, rows, h), lambda p, i: (0, i * (1 - p), 0)),
                full2((h, h2w)), full2((h2w,)),
                full2((h2w, h)), full2((h,)),
                pl.BlockSpec((1, 1, rows), lambda p, i: (i * (1 - p), 0, 0)),
                full2((g, out_dim)), full2((h, h2w)),
                full2((out_dim, h2w)), full2((h2w,)),
                full2((h2w,)), full2((h2w,)),
                full2((h2w, h)), full2((h,))],
      out_specs=pl.BlockSpec((rows, h), lambda p, i: (i * p, 0)),
      out_shape=jax.ShapeDtypeStruct((n, h), jnp.float32),
      scratch_shapes=[pltpu.VMEM((n, h2w), jnp.float32),
                      pltpu.VMEM((2, h2w), jnp.float32)],
  )(agg1, W1_1, b1_1, W2_1, b2_1, batch3, y_pred, wf1a, wf1b, bf1,
    gamma, beta, Wf2, bf2)

  return node_score


# speedup vs baseline: 1.6241x; 1.0204x over previous
import functools

import jax
import jax.numpy as jnp
from jax import lax
from jax.experimental import pallas as pl
from jax.experimental.pallas import tpu as pltpu
from jax.experimental.pallas import tpu_sc as plsc

_NC = 2
_NS = 16
_ROWS = 5000


_CHUNK = 80
_SBS = 25
_RING = 3
_LOOK = 2
_PAD_ROWS = 8


def _pad_edges(e, nw):
  epw = e // nw
  blk = _CHUNK * _SBS
  return -(-epw // blk) * blk


def _make_seg_sum(n, e, h):
  nw = _NC * _NS
  epw = _pad_edges(e, nw)
  chunk = _CHUNK
  steps = epw // chunk
  sbs = _SBS
  assert epw % (chunk * sbs) == 0
  spl = -(-n // _NS) // 8 * 8 + 8
  last = n - (_NS - 1) * spl
  assert last > 0 and last % 8 == 0

  nsb = steps // sbs

  def body(h_hbm, src_hbm, dst_hbm, zeros_hbm, out_hbm,
           acc, src_v, dst_v, *bufs_and_sems):
    cid = lax.axis_index("c")
    sid = lax.axis_index("s")
    row0 = pl.multiple_of(sid * spl, 8)

    def seed(rows):
      @pl.when(cid == 0)
      def _():
        pltpu.sync_copy(h_hbm.at[pl.ds(row0, rows)],
                        acc.at[pl.ds(row0, rows)])

      @pl.when(cid != 0)
      def _():
        pltpu.sync_copy(zeros_hbm.at[pl.ds(row0, rows)],
                        acc.at[pl.ds(row0, rows)])

    @pl.when(sid < _NS - 1)
    def _():
      seed(spl)

    @pl.when(sid == _NS - 1)
    def _():
      seed(last)

    plsc.subcore_barrier()

    wid = sid * _NC + cid
    ring, look = _RING, _LOOK
    bufs = bufs_and_sems[:ring]
    sems = bufs_and_sems[ring:ring + ring]
    sem_i = bufs_and_sems[ring + ring]

    def idx_row(ref, i):
      return ref.at[lax.rem(lax.div(i, sbs), 2), lax.rem(i, sbs)]

    def start_gather(i, k):
      pltpu.async_copy(h_hbm.at[idx_row(src_v, i)], bufs[k], sems[k])

    def wait_gather(i, k):
      pltpu.make_async_copy(h_hbm.at[idx_row(src_v, i)], bufs[k],
                            sems[k]).wait()

    def start_scatter(i, k):
      pltpu.async_copy(bufs[k], acc.at[idx_row(dst_v, i)], sems[k], add=True)

    def wait_scatter(i, k):
      pltpu.make_async_copy(bufs[k], acc.at[idx_row(dst_v, i)],
                            sems[k]).wait()

    def wait_staging(pn):
      pltpu.make_async_copy(src_hbm.at[wid, 0], src_v.at[pn], sem_i).wait()
      pltpu.make_async_copy(dst_hbm.at[wid, 0], dst_v.at[pn], sem_i).wait()

    pltpu.sync_copy(src_hbm.at[wid, 0], src_v.at[0])
    pltpu.sync_copy(dst_hbm.at[wid, 0], dst_v.at[0])
    for c in range(look):
      start_gather(c, c % ring)

    def step(i, c2):
      m = lax.rem(i, ring)
      blk = lax.div(i, sbs)
      for k in range(ring):
        @pl.when(m == k)
        def _(k=k):
          wait_gather(i, k)
          start_scatter(i, k)

          @pl.when((lax.rem(i, sbs) == 0) & (blk + 1 < nsb))
          def _():
            pn = lax.rem(blk + 1, 2)
            pltpu.async_copy(src_hbm.at[wid, blk + 1], src_v.at[pn], sem_i)
            pltpu.async_copy(dst_hbm.at[wid, blk + 1], dst_v.at[pn], sem_i)

          kn = (k + look) % ring

          @pl.when(i >= ring - look)
          def _():
            wait_scatter(i - (ring - look), kn)

          @pl.when(i + look < steps)
          def _():
            @pl.when(lax.rem(i + look, sbs) == 0)
            def _():
              wait_staging(lax.rem(lax.div(i + look, sbs), 2))

            start_gather(i + look, kn)

      return c2

    lax.fori_loop(0, steps, step, 0)
    for c in range(steps - (ring - look), steps):
      wait_scatter(c, c % ring)
    plsc.subcore_barrier()

    def writeout(rows):
      pltpu.sync_copy(acc.at[pl.ds(row0, rows)],
                      out_hbm.at[cid, pl.ds(row0, rows)])

    @pl.when(sid < _NS - 1)
    def _():
      writeout(spl)

    @pl.when(sid == _NS - 1)
    def _():
      writeout(last)

  return pl.kernel(
      body,
      out_type=jax.ShapeDtypeStruct((_NC, n, h), jnp.float32),
      mesh=plsc.VectorSubcoreMesh(core_axis_name="c", subcore_axis_name="s",
                                  num_cores=_NC, num_subcores=_NS),
      scratch_types=(
          [pltpu.VMEM_SHARED((n + _PAD_ROWS, h), jnp.float32),
           pltpu.VMEM((2, sbs, chunk), jnp.int32),
           pltpu.VMEM((2, sbs, chunk), jnp.int32)]
          + [pltpu.VMEM((chunk, h), jnp.float32)] * _RING
          + [pltpu.SemaphoreType.DMA] * (_RING + 1)
      ),
  )


def _dotf(a, b):
  return jnp.dot(a, b, preferred_element_type=jnp.float32)


def _embed_body(x_ref, w_ref, b_ref, o_ref):
  o_ref[...] = _dotf(x_ref[...], w_ref[...]) + b_ref[...]


def _gin_body(agg_ref, w1_ref, b1_ref, w2_ref, b2_ref, o_ref):
  t = agg_ref[0] + agg_ref[1]
  t = jnp.maximum(_dotf(t, w1_ref[...]) + b1_ref[...], 0.0)
  o_ref[...] = _dotf(t, w2_ref[...]) + b2_ref[...]


def _fuse_fin_body(g, n, rows, agg_ref, w1_ref, b1_ref, w2_ref, b2_ref,
                   batch_ref, ypred_ref, wf1a_ref, wf1b_ref, bf1_ref,
                   gamma_ref, beta_ref, wf2_ref, bf2_ref,
                   o_ref, z_scr, sums_scr):
  p = pl.program_id(0)
  i = pl.program_id(1)

  @pl.when(p == 0)
  def _():
    t = agg_ref[0] + agg_ref[1]
    t = jnp.maximum(_dotf(t, w1_ref[...]) + b1_ref[...], 0.0)
    h2 = _dotf(t, w2_ref[...]) + b2_ref[...]
    yproj = _dotf(ypred_ref[...], wf1b_ref[...])
    b = batch_ref[0, 0, :]
    onehot = (b[:, None] ==
              lax.broadcasted_iota(jnp.int32, (b.shape[0], g), 1)
              ).astype(jnp.float32)
    z = _dotf(h2, wf1a_ref[...]) + _dotf(onehot, yproj) + bf1_ref[...]
    z_scr[pl.ds(i * rows, rows), :] = z

    @pl.when(i == 0)
    def _():
      sums_scr[...] = jnp.zeros_like(sums_scr)

    s1 = jnp.sum(z, axis=0)
    s2 = jnp.sum(z * z, axis=0)
    sums_scr[...] = sums_scr[...] + jnp.concatenate(
        [s1[None, :], s2[None, :]], axis=0)

  @pl.when(p == 1)
  def _():
    mean = sums_scr[0, :] / n
    var = sums_scr[1, :] / n - mean * mean
    scale = lax.rsqrt(var + 1e-5) * gamma_ref[...]
    z = z_scr[pl.ds(i * rows, rows), :]
    zn = (z - mean) * scale + beta_ref[...]
    zn = jnp.maximum(zn, 0.0)
    o = _dotf(zn, wf2_ref[...]) + bf2_ref[...]
    o_ref[...] = jax.nn.sigmoid(o)


def _full(shape):
  nd = len(shape)
  return pl.BlockSpec(shape, lambda i: (0,) * nd)


def kernel(x, edge_index, batch, y_pred, W_embed, b_embed,
           W1_0, b1_0, W2_0, b2_0, W1_1, b1_1, W2_1, b2_1,
           Wf1, bf1, gamma, beta, Wf2, bf2):
  n, d = x.shape
  h = W_embed.shape[1]
  e = edge_index.shape[1]
  g, out_dim = y_pred.shape
  h2w = 2 * h
  rows = _ROWS
  n_tiles = n // rows
  assert n % rows == 0

  nw = _NC * _NS
  epw_pad = _pad_edges(e, nw)
  pad = epw_pad - e // nw
  nsb = epw_pad // (_CHUNK * _SBS)

  def shard_edges(idx, pad_val):
    shards = idx.reshape(nw, e // nw)
    if pad:
      fill = jnp.broadcast_to(pad_val, (nw, pad))
      shards = jnp.concatenate([shards, fill], axis=1)
    return shards.reshape(nw, nsb, _SBS, _CHUNK)

  pad_iota = jnp.arange(pad, dtype=jnp.int32) % _PAD_ROWS
  src = shard_edges(edge_index[0], pad_iota)
  dst = shard_edges(edge_index[1], n + pad_iota)
  zeros = jnp.zeros((n, h), jnp.float32)
  batch3 = batch.reshape(n_tiles, 1, rows)
  wf1a = Wf1[:h]
  wf1b = Wf1[h:]

  seg_sum = _make_seg_sum(n, e, h)

  row_spec = pl.BlockSpec((rows, h), lambda i: (i, 0))
  agg_spec = pl.BlockSpec((_NC, rows, h), lambda i: (0, i, 0))

  h0 = pl.pallas_call(
      _embed_body,
      grid=(n_tiles,),
      in_specs=[pl.BlockSpec((rows, d), lambda i: (i, 0)),
                _full((d, h)), _full((h,))],
      out_specs=row_spec,
      out_shape=jax.ShapeDtypeStruct((n, h), jnp.float32),
  )(x, W_embed, b_embed)

  agg0 = seg_sum(h0, src, dst, zeros)

  h1 = pl.pallas_call(
      _gin_body,
      grid=(n_tiles,),
      in_specs=[agg_spec, _full((h, h2w)), _full((h2w,)),
                _full((h2w, h)), _full((h,))],
      out_specs=row_spec,
      out_shape=jax.ShapeDtypeStruct((n, h), jnp.float32),
  )(agg0, W1_0, b1_0, W2_0, b2_0)

  agg1 = seg_sum(h1, src, dst, zeros)

  def full2(shape):
    nd = len(shape)
    return pl.BlockSpec(shape, lambda p, i: (0,) * nd)

  node_score = pl.pallas_call(
      functools.partial(_fuse_fin_body, g, float(n), rows),
      grid=(2, n_tiles),
      in_specs=[pl.BlockSpec((_NC, rows, h), lambda p, i: (0, i * (1 - p), 0)),
                full2((h, h2w)), full2((h2w,)),
                full2((h2w, h)), full2((h,)),
                pl.BlockSpec((1, 1, rows), lambda p, i: (i * (1 - p), 0, 0)),
                full2((g, out_dim)), full2((h, h2w)),
                full2((out_dim, h2w)), full2((h2w,)),
                full2((h2w,)), full2((h2w,)),
                full2((h2w, h)), full2((h,))],
      out_specs=pl.BlockSpec((rows, h), lambda p, i: (i * p, 0)),
      out_shape=jax.ShapeDtypeStruct((n, h), jnp.float32),
      scratch_shapes=[pltpu.VMEM((n, h2w), jnp.float32),
                      pltpu.VMEM((2, h2w), jnp.float32)],
  )(agg1, W1_1, b1_1, W2_1, b2_1, batch3, y_pred, wf1a, wf1b, bf1,
    gamma, beta, Wf2, bf2)

  return node_score
